# baseline (speedup 1.0000x reference)
import functools

import jax
import jax.numpy as jnp
from jax.experimental import pallas as pl

TOPK = 16
LANE_TILE = 256


def _topk_mask_kernel(corr_ref, out_ref, *, topk):
    corr = corr_ref[0]
    m = jnp.abs(corr)
    rowsf = jax.lax.broadcasted_iota(jnp.int32, m.shape, 0).astype(jnp.float32)
    big = jnp.float32(1e9)
    mwork = m
    for _ in range(topk):
        v = jnp.max(mwork, axis=0, keepdims=True)
        rowm = jnp.where(mwork == v, rowsf, big)
        rpick = jnp.min(rowm, axis=0, keepdims=True)
        mwork = mwork + jnp.where(rowsf == rpick, -big, 0.0)
    out_ref[0] = jnp.where(mwork < -big * 0.5, corr, 0.0)


def kernel(x):
    B, L, C = x.shape
    X_freq = jnp.fft.rfft(x, axis=1)
    AC_freq = X_freq * jnp.conj(X_freq)
    corr_time = jnp.fft.irfft(AC_freq, n=L, axis=1)
    k = min(TOPK, L)
    grid = (B, C // LANE_TILE)
    return pl.pallas_call(
        functools.partial(_topk_mask_kernel, topk=k),
        grid=grid,
        in_specs=[pl.BlockSpec((1, L, LANE_TILE), lambda b, j: (b, 0, j))],
        out_specs=pl.BlockSpec((1, L, LANE_TILE), lambda b, j: (b, 0, j)),
        out_shape=jax.ShapeDtypeStruct((B, L, C), corr_time.dtype),
        input_output_aliases={0: 0},
    )(corr_time)


# speedup vs baseline: 1.0116x; 1.0116x over previous
import functools

import jax
import jax.numpy as jnp
from jax.experimental import pallas as pl

TOPK = 16
LANE_TILE = 256


def _topk_mask_kernel(corr_ref, out_ref, *, topk):
    corr = corr_ref[0]
    m = jnp.abs(corr)
    rowsf = jax.lax.broadcasted_iota(jnp.int32, m.shape, 0).astype(jnp.float32)
    big = jnp.float32(1e9)
    mwork = m
    for _ in range(topk):
        v = jnp.max(mwork, axis=0, keepdims=True)
        rowm = jnp.where(mwork == v, rowsf, big)
        rpick = jnp.min(rowm, axis=0, keepdims=True)
        mwork = jnp.where(rowsf == rpick, -big, mwork)
    out_ref[0] = jnp.where(mwork < -big * 0.5, corr, 0.0)


def kernel(x):
    B, L, C = x.shape
    X_freq = jnp.fft.rfft(x, axis=1)
    AC_freq = X_freq * jnp.conj(X_freq)
    corr_time = jnp.fft.irfft(AC_freq, n=L, axis=1)
    k = min(TOPK, L)
    grid = (B, C // LANE_TILE)
    return pl.pallas_call(
        functools.partial(_topk_mask_kernel, topk=k),
        grid=grid,
        in_specs=[pl.BlockSpec((1, L, LANE_TILE), lambda b, j: (b, 0, j))],
        out_specs=pl.BlockSpec((1, L, LANE_TILE), lambda b, j: (b, 0, j)),
        out_shape=jax.ShapeDtypeStruct((B, L, C), corr_time.dtype),
        input_output_aliases={0: 0},
    )(corr_time)


# speedup vs baseline: 1.0118x; 1.0001x over previous
import functools

import jax
import jax.numpy as jnp
from jax.experimental import pallas as pl
from jax.experimental.pallas import tpu as pltpu

TOPK = 16
LANE_TILE = 256


def _topk_mask_kernel(corr_ref, out_ref, *, topk):
    corr = corr_ref[0]
    m = jnp.abs(corr)
    rowsf = jax.lax.broadcasted_iota(jnp.int32, m.shape, 0).astype(jnp.float32)
    big = jnp.float32(1e9)
    mwork = m
    for _ in range(topk):
        v = jnp.max(mwork, axis=0, keepdims=True)
        rowm = jnp.where(mwork == v, rowsf, big)
        rpick = jnp.min(rowm, axis=0, keepdims=True)
        mwork = jnp.where(rowsf == rpick, -big, mwork)
    out_ref[0] = jnp.where(mwork < -big * 0.5, corr, 0.0)


def kernel(x):
    B, L, C = x.shape
    X_freq = jnp.fft.rfft(x, axis=1)
    AC_freq = X_freq * jnp.conj(X_freq)
    corr_time = jnp.fft.irfft(AC_freq, n=L, axis=1)
    k = min(TOPK, L)
    grid = (B, C // LANE_TILE)
    return pl.pallas_call(
        functools.partial(_topk_mask_kernel, topk=k),
        grid=grid,
        in_specs=[pl.BlockSpec((1, L, LANE_TILE), lambda b, j: (b, 0, j))],
        out_specs=pl.BlockSpec((1, L, LANE_TILE), lambda b, j: (b, 0, j)),
        out_shape=jax.ShapeDtypeStruct((B, L, C), corr_time.dtype),
        input_output_aliases={0: 0},
        compiler_params=pltpu.CompilerParams(
            dimension_semantics=("parallel", "parallel")),
    )(corr_time)
